# Initial kernel scaffold; baseline (speedup 1.0000x reference)
import jax
import jax.numpy as jnp
from jax.experimental import pallas as pl


def kernel(embeddings_int, embeddings_pop, q, b, user, item_p, item_n, mask, edge_index):
    raise NotImplementedError("write your pallas kernel here")


# speedup vs baseline: 4.8143x; 4.8143x over previous
import jax
import jax.numpy as jnp
from jax import lax
from jax.experimental import pallas as pl
from jax.experimental.pallas import tpu as pltpu
from jax.experimental.pallas import tpu_sc as plsc

_N_USER = 6000
_N_ITEM = 4000
_N = 10000
_NP = 10240
_D = 128
_E = 320000
_EC = 64
_EDGE_CHUNKS = 314
_EP = 321536
_ROWS_PER_TILE = 640
_TRI = 16384
_TRI_CHUNKS = 16


def _rsqrt16(x):
    i = lax.bitcast_convert_type(x, jnp.int32)
    i = jnp.int32(0x5F3759DF) - (i >> 1)
    y = lax.bitcast_convert_type(i, jnp.float32)
    for _ in range(3):
        y = y * (jnp.float32(1.5) - jnp.float32(0.5) * x * y * y)
    return y


def _splat(ref, pos):
    return plsc.load_gather(ref, [jnp.full((16,), pos, jnp.int32)])


def _sc_body(emb2, srcp, dstp, uf, ipf, inf_, q_hbm, b_hbm,
             r4, qb4, g2, s2,
             acc_sh, degs_sh, degd_sh,
             big_v, a_v, b_v, eidx_v, tidx_v, qidx_v,
             rout_v, qbout_v, zrow_v, ones_v,
             sem0, sem1):
    c = lax.axis_index("c")
    s = lax.axis_index("s")
    r0 = s * _ROWS_PER_TILE
    toff = s * (_EDGE_CHUNKS * _EC)
    zero16 = jnp.zeros((16,), jnp.float32)
    one16 = jnp.ones((16,), jnp.float32)

    for r in range(16):
        for j in range(8):
            zrow_v[r, pl.ds(16 * j, 16)] = zero16
    for j in range(4):
        ones_v[pl.ds(16 * j, 16)] = one16

    @pl.loop(0, _ROWS_PER_TILE // 16)
    def _zero_deg(i):
        a_v[pl.ds(i * 16, 16)] = zero16
        b_v[pl.ds(i * 16, 16)] = zero16

    pltpu.sync_copy(a_v, degs_sh.at[pl.ds(r0, _ROWS_PER_TILE)])
    pltpu.sync_copy(b_v, degd_sh.at[pl.ds(r0, _ROWS_PER_TILE)])

    @pl.loop(0, _ROWS_PER_TILE // 16)
    def _zacc(i):
        pltpu.sync_copy(zrow_v, acc_sh.at[pl.ds(r0 + i * 16, 16)])

    plsc.subcore_barrier()

    @pl.loop(0, _EDGE_CHUNKS)
    def _hist(k):
        off = toff + k * _EC
        pltpu.sync_copy(srcp.at[pl.ds(off, _EC)], eidx_v.at[0])
        pltpu.sync_copy(dstp.at[pl.ds(off, _EC)], eidx_v.at[1])
        pltpu.sync_copy(ones_v, degs_sh.at[eidx_v.at[0]], add=True)
        pltpu.sync_copy(ones_v, degd_sh.at[eidx_v.at[1]], add=True)

    plsc.subcore_barrier()

    pltpu.sync_copy(degs_sh.at[pl.ds(r0, _ROWS_PER_TILE)], a_v)
    pltpu.sync_copy(degd_sh.at[pl.ds(r0, _ROWS_PER_TILE)], b_v)

    @pl.loop(0, _ROWS_PER_TILE // 16)
    def _rs(i):
        x = a_v[pl.ds(i * 16, 16)]
        a_v[pl.ds(i * 16, 16)] = _rsqrt16(jnp.maximum(x, jnp.float32(1.0)))
        y = b_v[pl.ds(i * 16, 16)]
        b_v[pl.ds(i * 16, 16)] = _rsqrt16(jnp.maximum(y, jnp.float32(1.0)))

    nreal = jnp.minimum(_ROWS_PER_TILE, jnp.maximum(0, _N - r0))
    nch = nreal // 16

    @pl.loop(0, nch)
    def _g0(i):
        rr = r0 + i * 16
        pltpu.sync_copy(emb2.at[c, pl.ds(rr, 16)], big_v.at[0, pl.ds(0, 16)])
        for r in range(16):
            av = _splat(a_v, i * 16 + r)
            for j in range(8):
                big_v[1, r, pl.ds(16 * j, 16)] = (
                    big_v[0, r, pl.ds(16 * j, 16)] * av)
        pltpu.sync_copy(big_v.at[1, pl.ds(0, 16)], g2.at[c, pl.ds(rr, 16)])

    @pl.when(s == 15)
    def _():
        @pl.loop(0, (_NP - _N) // 16)
        def _zpad(i):
            pltpu.sync_copy(zrow_v, g2.at[c, pl.ds(_N + i * 16, 16)])

    plsc.subcore_barrier()

    def _edge_pass():
        @pl.loop(0, _EDGE_CHUNKS)
        def _edges(k):
            off = toff + k * _EC
            pltpu.sync_copy(srcp.at[pl.ds(off, _EC)], eidx_v.at[0])
            pltpu.sync_copy(dstp.at[pl.ds(off, _EC)], eidx_v.at[1])
            pltpu.async_copy(g2.at[c].at[eidx_v.at[0]], big_v.at[0],
                             sem0).wait()
            pltpu.sync_copy(big_v.at[0], acc_sh.at[eidx_v.at[1]], add=True)

    _edge_pass()
    plsc.subcore_barrier()

    @pl.loop(0, nch)
    def _np1(i):
        rr = r0 + i * 16
        pltpu.sync_copy(acc_sh.at[pl.ds(rr, 16)], big_v.at[0, pl.ds(0, 16)])
        pltpu.sync_copy(emb2.at[c, pl.ds(rr, 16)], big_v.at[1, pl.ds(0, 16)])
        for r in range(16):
            bv = _splat(b_v, i * 16 + r)
            av = _splat(a_v, i * 16 + r)
            for j in range(8):
                h1 = big_v[0, r, pl.ds(16 * j, 16)] * bv
                big_v[1, r, pl.ds(16 * j, 16)] = (
                    big_v[1, r, pl.ds(16 * j, 16)] + h1)
                big_v[0, r, pl.ds(16 * j, 16)] = h1 * av
        pltpu.sync_copy(big_v.at[1, pl.ds(0, 16)], s2.at[c, pl.ds(rr, 16)])
        pltpu.sync_copy(big_v.at[0, pl.ds(0, 16)], g2.at[c, pl.ds(rr, 16)])
        pltpu.sync_copy(zrow_v, acc_sh.at[pl.ds(rr, 16)])

    plsc.subcore_barrier()

    _edge_pass()
    plsc.subcore_barrier()

    third = jnp.float32(1.0 / 3.0)

    @pl.loop(0, nch)
    def _np2(i):
        rr = r0 + i * 16
        pltpu.sync_copy(acc_sh.at[pl.ds(rr, 16)], big_v.at[0, pl.ds(0, 16)])
        pltpu.sync_copy(s2.at[c, pl.ds(rr, 16)], big_v.at[1, pl.ds(0, 16)])
        for r in range(16):
            bv = _splat(b_v, i * 16 + r)
            for j in range(8):
                big_v[1, r, pl.ds(16 * j, 16)] = (
                    big_v[1, r, pl.ds(16 * j, 16)]
                    + big_v[0, r, pl.ds(16 * j, 16)] * bv) * third
        pltpu.sync_copy(big_v.at[1, pl.ds(0, 16)], g2.at[c, pl.ds(rr, 16)])

    plsc.subcore_barrier()

    @pl.loop(0, _TRI_CHUNKS)
    def _dots(ch):
        t0 = s * (_TRI_CHUNKS * 64) + ch * 64
        pltpu.sync_copy(uf.at[pl.ds(t0, 64)], tidx_v.at[0])
        pltpu.sync_copy(ipf.at[pl.ds(t0, 64)], tidx_v.at[1])
        pltpu.sync_copy(inf_.at[pl.ds(t0, 64)], tidx_v.at[2])
        pltpu.async_copy(g2.at[c].at[tidx_v.at[0]], big_v.at[0], sem0).wait()
        pltpu.async_copy(g2.at[c].at[tidx_v.at[1]], big_v.at[1], sem0).wait()
        pltpu.async_copy(g2.at[c].at[tidx_v.at[2]], big_v.at[2], sem1).wait()

        @pl.loop(0, 64)
        def _row(r):
            pacc = jnp.zeros((16,), jnp.float32)
            nacc = jnp.zeros((16,), jnp.float32)
            for j in range(8):
                u = big_v[0, r, pl.ds(16 * j, 16)]
                pacc = pacc + u * big_v[1, r, pl.ds(16 * j, 16)]
                nacc = nacc + u * big_v[2, r, pl.ds(16 * j, 16)]
            rout_v[0, r, pl.ds(0, 16)] = pacc
            rout_v[1, r, pl.ds(0, 16)] = nacc

        pltpu.sync_copy(rout_v.at[0], r4.at[2 * c, pl.ds(t0, 64)])
        pltpu.sync_copy(rout_v.at[1], r4.at[2 * c + 1, pl.ds(t0, 64)])

        @pl.when(c == 1)
        def _():
            for j in range(4):
                sl = pl.ds(16 * j, 16)
                qidx_v[0, sl] = tidx_v[1, sl] - _N_USER
                qidx_v[1, sl] = tidx_v[2, sl] - _N_USER
            pltpu.async_copy(q_hbm.at[qidx_v.at[0]], qbout_v.at[0],
                             sem0).wait()
            pltpu.async_copy(q_hbm.at[qidx_v.at[1]], qbout_v.at[1],
                             sem0).wait()
            pltpu.async_copy(b_hbm.at[qidx_v.at[0]], qbout_v.at[2],
                             sem1).wait()
            pltpu.async_copy(b_hbm.at[qidx_v.at[1]], qbout_v.at[3],
                             sem1).wait()
            for z in range(4):
                pltpu.sync_copy(qbout_v.at[z], qb4.at[z, pl.ds(t0, 64)])


def _make_sc_call():
    mesh = plsc.VectorSubcoreMesh(core_axis_name="c", subcore_axis_name="s")
    out_type = (
        jax.ShapeDtypeStruct((4, _TRI, 16), jnp.float32),
        jax.ShapeDtypeStruct((4, _TRI), jnp.float32),
        jax.ShapeDtypeStruct((2, _NP, _D), jnp.float32),
        jax.ShapeDtypeStruct((2, _NP, _D), jnp.float32),
    )
    scratch = [
        pltpu.VMEM_SHARED((_NP, _D), jnp.float32),
        pltpu.VMEM_SHARED((_NP,), jnp.float32),
        pltpu.VMEM_SHARED((_NP,), jnp.float32),
        pltpu.VMEM((3, _EC, _D), jnp.float32),
        pltpu.VMEM((_ROWS_PER_TILE,), jnp.float32),
        pltpu.VMEM((_ROWS_PER_TILE,), jnp.float32),
        pltpu.VMEM((2, _EC), jnp.int32),
        pltpu.VMEM((3, _EC), jnp.int32),
        pltpu.VMEM((2, _EC), jnp.int32),
        pltpu.VMEM((2, _EC, 16), jnp.float32),
        pltpu.VMEM((4, _EC), jnp.float32),
        pltpu.VMEM((16, _D), jnp.float32),
        pltpu.VMEM((_EC,), jnp.float32),
        pltpu.SemaphoreType.DMA,
        pltpu.SemaphoreType.DMA,
    ]
    return pl.kernel(_sc_body, out_type=out_type, mesh=mesh,
                     scratch_types=scratch,
                     compiler_params=pltpu.CompilerParams(
                         needs_layout_passes=False))


def _loss_body(rr_ref, qq_ref, mm_ref, out_ref):
    def logsig(x):
        return jnp.minimum(x, 0.0) - jnp.log(1.0 + jnp.exp(-jnp.abs(x)))

    def softplus(x):
        return jnp.maximum(x, 0.0) + jnp.log(1.0 + jnp.exp(-jnp.abs(x)))

    p_int = jnp.sum(rr_ref[0], axis=-1)
    n_int = jnp.sum(rr_ref[1], axis=-1)
    p_pop = jnp.sum(rr_ref[2], axis=-1)
    n_pop = jnp.sum(rr_ref[3], axis=-1)
    mf = mm_ref[...]
    loss_int = -jnp.mean(mf * logsig(p_int - n_int))
    loss_pop = (-jnp.mean(mf * logsig(n_pop - p_pop))
                + jnp.mean((1.0 - mf) * logsig(p_pop - n_pop)))
    pop_p = softplus(qq_ref[0]) + softplus(qq_ref[2])
    pop_n = softplus(qq_ref[1]) + softplus(qq_ref[3])
    p_tot = p_int + p_pop
    n_tot = n_int + n_pop
    loss_tide = -jnp.mean(logsig(jnp.tanh(pop_p) * p_tot
                                 - jnp.tanh(pop_n) * n_tot))
    total = 0.5 * loss_int + 0.5 * loss_pop + 0.2 * loss_tide
    out_ref[...] = jnp.full((1, 1), total, jnp.float32)


@jax.jit
def kernel(embeddings_int, embeddings_pop, q, b, user, item_p, item_n, mask,
           edge_index):
    emb2 = jnp.stack([embeddings_int, embeddings_pop])
    pad = jnp.full((2, _EP - _E), _N, jnp.int32)
    ep = jnp.concatenate([edge_index.astype(jnp.int32), pad], axis=1)
    srcp = ep[0]
    dstp = ep[1]
    uf = user.reshape(-1).astype(jnp.int32)
    ipf = (item_p.reshape(-1) + _N_USER).astype(jnp.int32)
    inf_ = (item_n.reshape(-1) + _N_USER).astype(jnp.int32)

    r4, qb4, _, _ = _make_sc_call()(emb2, srcp, dstp, uf, ipf, inf_, q, b)

    rr = r4.reshape(4, 128, 128, 16)
    qq = qb4.reshape(4, 128, 128)
    mm = mask.reshape(-1).astype(jnp.float32).reshape(128, 128)
    out = pl.pallas_call(
        _loss_body,
        out_shape=jax.ShapeDtypeStruct((1, 1), jnp.float32),
    )(rr, qq, mm)
    return out[0, 0]


# speedup vs baseline: 5.1796x; 1.0759x over previous
import jax
import jax.numpy as jnp
from jax import lax
from jax.experimental import pallas as pl
from jax.experimental.pallas import tpu as pltpu
from jax.experimental.pallas import tpu_sc as plsc

_N_USER = 6000
_N_ITEM = 4000
_N = 10000
_NP = 10240
_D = 128
_E = 320000
_EC = 64
_BLK = 3
_EDGE_CHUNKS = 321
_EP = 328704
_ROWS_PER_TILE = 640
_TRI = 16384
_TRI_CHUNKS = 16


def _rsqrt16(x):
    i = lax.bitcast_convert_type(x, jnp.int32)
    i = jnp.int32(0x5F3759DF) - (i >> 1)
    y = lax.bitcast_convert_type(i, jnp.float32)
    for _ in range(3):
        y = y * (jnp.float32(1.5) - jnp.float32(0.5) * x * y * y)
    return y


def _splat(ref, pos):
    return plsc.load_gather(ref, [jnp.full((16,), pos, jnp.int32)])


def _sc_body(emb2, srcp, dstp, uf, ipf, inf_, q_hbm, b_hbm,
             r4, qb4, g2, s2,
             acc_sh, degs_sh, degd_sh,
             big_v, a_v, b_v, esrc_v, edst_v, tidx_v, qidx_v,
             rout_v, qbout_v, zrow_v, ones_v,
             sem0, sem1, sem2):
    c = lax.axis_index("c")
    s = lax.axis_index("s")
    r0 = s * _ROWS_PER_TILE
    toff = s * (_EDGE_CHUNKS * _EC)
    zero16 = jnp.zeros((16,), jnp.float32)
    one16 = jnp.ones((16,), jnp.float32)

    for r in range(16):
        for j in range(8):
            zrow_v[r, pl.ds(16 * j, 16)] = zero16
    for j in range(4):
        ones_v[pl.ds(16 * j, 16)] = one16

    @pl.loop(0, _ROWS_PER_TILE // 16)
    def _zero_deg(i):
        a_v[pl.ds(i * 16, 16)] = zero16
        b_v[pl.ds(i * 16, 16)] = zero16

    pltpu.sync_copy(a_v, degs_sh.at[pl.ds(r0, _ROWS_PER_TILE)])
    pltpu.sync_copy(b_v, degd_sh.at[pl.ds(r0, _ROWS_PER_TILE)])

    @pl.loop(0, _ROWS_PER_TILE // 16)
    def _zacc(i):
        pltpu.sync_copy(zrow_v, acc_sh.at[pl.ds(r0 + i * 16, 16)])

    plsc.subcore_barrier()

    def _load_idx_block(bk):
        off = toff + bk * (_BLK * _EC)
        hi = []
        for j in range(_BLK):
            hi.append(pltpu.async_copy(
                srcp.at[pl.ds(off + j * _EC, _EC)], esrc_v.at[j], sem2))
            hi.append(pltpu.async_copy(
                dstp.at[pl.ds(off + j * _EC, _EC)], edst_v.at[j], sem2))
        for h in hi:
            h.wait()

    @pl.loop(0, _EDGE_CHUNKS // _BLK)
    def _hist(bk):
        _load_idx_block(bk)
        hs = []
        for j in range(_BLK):
            hs.append(pltpu.async_copy(
                ones_v, degs_sh.at[esrc_v.at[j]], sem1, add=True))
            hs.append(pltpu.async_copy(
                ones_v, degd_sh.at[edst_v.at[j]], sem1, add=True))
        for h in hs:
            h.wait()

    plsc.subcore_barrier()

    pltpu.sync_copy(degs_sh.at[pl.ds(r0, _ROWS_PER_TILE)], a_v)
    pltpu.sync_copy(degd_sh.at[pl.ds(r0, _ROWS_PER_TILE)], b_v)

    @pl.loop(0, _ROWS_PER_TILE // 16)
    def _rs(i):
        x = a_v[pl.ds(i * 16, 16)]
        a_v[pl.ds(i * 16, 16)] = _rsqrt16(jnp.maximum(x, jnp.float32(1.0)))
        y = b_v[pl.ds(i * 16, 16)]
        b_v[pl.ds(i * 16, 16)] = _rsqrt16(jnp.maximum(y, jnp.float32(1.0)))

    nreal = jnp.minimum(_ROWS_PER_TILE, jnp.maximum(0, _N - r0))
    nch = nreal // 16

    @pl.loop(0, nch)
    def _g0(i):
        rr = r0 + i * 16
        pltpu.sync_copy(emb2.at[c, pl.ds(rr, 16)], big_v.at[0, pl.ds(0, 16)])
        for r in range(16):
            av = _splat(a_v, i * 16 + r)
            for j in range(8):
                big_v[1, r, pl.ds(16 * j, 16)] = (
                    big_v[0, r, pl.ds(16 * j, 16)] * av)
        pltpu.sync_copy(big_v.at[1, pl.ds(0, 16)], g2.at[c, pl.ds(rr, 16)])

    @pl.when(s == 15)
    def _():
        @pl.loop(0, (_NP - _N) // 16)
        def _zpad(i):
            pltpu.sync_copy(zrow_v, g2.at[c, pl.ds(_N + i * 16, 16)])

    plsc.subcore_barrier()

    def _edge_pass():
        @pl.loop(0, _EDGE_CHUNKS // _BLK)
        def _edges(bk):
            _load_idx_block(bk)
            hg = [pltpu.async_copy(g2.at[c].at[esrc_v.at[j]],
                                   big_v.at[j], sem0)
                  for j in range(_BLK)]
            for h in hg:
                h.wait()
            hsct = [pltpu.async_copy(big_v.at[j],
                                     acc_sh.at[edst_v.at[j]], sem1, add=True)
                    for j in range(_BLK)]
            for h in hsct:
                h.wait()

    _edge_pass()
    plsc.subcore_barrier()

    @pl.loop(0, nch)
    def _np1(i):
        rr = r0 + i * 16
        pltpu.sync_copy(acc_sh.at[pl.ds(rr, 16)], big_v.at[0, pl.ds(0, 16)])
        pltpu.sync_copy(emb2.at[c, pl.ds(rr, 16)], big_v.at[1, pl.ds(0, 16)])
        for r in range(16):
            bv = _splat(b_v, i * 16 + r)
            av = _splat(a_v, i * 16 + r)
            for j in range(8):
                h1 = big_v[0, r, pl.ds(16 * j, 16)] * bv
                big_v[1, r, pl.ds(16 * j, 16)] = (
                    big_v[1, r, pl.ds(16 * j, 16)] + h1)
                big_v[0, r, pl.ds(16 * j, 16)] = h1 * av
        pltpu.sync_copy(big_v.at[1, pl.ds(0, 16)], s2.at[c, pl.ds(rr, 16)])
        pltpu.sync_copy(big_v.at[0, pl.ds(0, 16)], g2.at[c, pl.ds(rr, 16)])
        pltpu.sync_copy(zrow_v, acc_sh.at[pl.ds(rr, 16)])

    plsc.subcore_barrier()

    _edge_pass()
    plsc.subcore_barrier()

    third = jnp.float32(1.0 / 3.0)

    @pl.loop(0, nch)
    def _np2(i):
        rr = r0 + i * 16
        pltpu.sync_copy(acc_sh.at[pl.ds(rr, 16)], big_v.at[0, pl.ds(0, 16)])
        pltpu.sync_copy(s2.at[c, pl.ds(rr, 16)], big_v.at[1, pl.ds(0, 16)])
        for r in range(16):
            bv = _splat(b_v, i * 16 + r)
            for j in range(8):
                big_v[1, r, pl.ds(16 * j, 16)] = (
                    big_v[1, r, pl.ds(16 * j, 16)]
                    + big_v[0, r, pl.ds(16 * j, 16)] * bv) * third
        pltpu.sync_copy(big_v.at[1, pl.ds(0, 16)], g2.at[c, pl.ds(rr, 16)])

    plsc.subcore_barrier()

    @pl.loop(0, _TRI_CHUNKS)
    def _dots(ch):
        t0 = s * (_TRI_CHUNKS * 64) + ch * 64
        pltpu.sync_copy(uf.at[pl.ds(t0, 64)], tidx_v.at[0])
        pltpu.sync_copy(ipf.at[pl.ds(t0, 64)], tidx_v.at[1])
        pltpu.sync_copy(inf_.at[pl.ds(t0, 64)], tidx_v.at[2])
        hg = [pltpu.async_copy(g2.at[c].at[tidx_v.at[z]], big_v.at[z], sem0)
              for z in range(3)]

        @pl.when(c == 1)
        def _():
            for j in range(4):
                sl = pl.ds(16 * j, 16)
                qidx_v[0, sl] = tidx_v[1, sl] - _N_USER
                qidx_v[1, sl] = tidx_v[2, sl] - _N_USER
            hq = [pltpu.async_copy(q_hbm.at[qidx_v.at[0]], qbout_v.at[0],
                                   sem2),
                  pltpu.async_copy(q_hbm.at[qidx_v.at[1]], qbout_v.at[1],
                                   sem2),
                  pltpu.async_copy(b_hbm.at[qidx_v.at[0]], qbout_v.at[2],
                                   sem2),
                  pltpu.async_copy(b_hbm.at[qidx_v.at[1]], qbout_v.at[3],
                                   sem2)]
            for h in hq:
                h.wait()
            for z in range(4):
                pltpu.sync_copy(qbout_v.at[z], qb4.at[z, pl.ds(t0, 64)])

        for h in hg:
            h.wait()

        @pl.loop(0, 64)
        def _row(r):
            pacc = jnp.zeros((16,), jnp.float32)
            nacc = jnp.zeros((16,), jnp.float32)
            for j in range(8):
                u = big_v[0, r, pl.ds(16 * j, 16)]
                pacc = pacc + u * big_v[1, r, pl.ds(16 * j, 16)]
                nacc = nacc + u * big_v[2, r, pl.ds(16 * j, 16)]
            rout_v[0, r, pl.ds(0, 16)] = pacc
            rout_v[1, r, pl.ds(0, 16)] = nacc

        pltpu.sync_copy(rout_v.at[0], r4.at[2 * c, pl.ds(t0, 64)])
        pltpu.sync_copy(rout_v.at[1], r4.at[2 * c + 1, pl.ds(t0, 64)])


def _make_sc_call():
    mesh = plsc.VectorSubcoreMesh(core_axis_name="c", subcore_axis_name="s")
    out_type = (
        jax.ShapeDtypeStruct((4, _TRI, 16), jnp.float32),
        jax.ShapeDtypeStruct((4, _TRI), jnp.float32),
        jax.ShapeDtypeStruct((2, _NP, _D), jnp.float32),
        jax.ShapeDtypeStruct((2, _NP, _D), jnp.float32),
    )
    scratch = [
        pltpu.VMEM_SHARED((_NP, _D), jnp.float32),
        pltpu.VMEM_SHARED((_NP,), jnp.float32),
        pltpu.VMEM_SHARED((_NP,), jnp.float32),
        pltpu.VMEM((_BLK, _EC, _D), jnp.float32),
        pltpu.VMEM((_ROWS_PER_TILE,), jnp.float32),
        pltpu.VMEM((_ROWS_PER_TILE,), jnp.float32),
        pltpu.VMEM((_BLK, _EC), jnp.int32),
        pltpu.VMEM((_BLK, _EC), jnp.int32),
        pltpu.VMEM((3, _EC), jnp.int32),
        pltpu.VMEM((2, _EC), jnp.int32),
        pltpu.VMEM((2, _EC, 16), jnp.float32),
        pltpu.VMEM((4, _EC), jnp.float32),
        pltpu.VMEM((16, _D), jnp.float32),
        pltpu.VMEM((_EC,), jnp.float32),
        pltpu.SemaphoreType.DMA,
        pltpu.SemaphoreType.DMA,
        pltpu.SemaphoreType.DMA,
    ]
    return pl.kernel(_sc_body, out_type=out_type, mesh=mesh,
                     scratch_types=scratch,
                     compiler_params=pltpu.CompilerParams(
                         needs_layout_passes=False))


def _loss_body(rr_ref, qq_ref, mm_ref, out_ref):
    def logsig(x):
        return jnp.minimum(x, 0.0) - jnp.log(1.0 + jnp.exp(-jnp.abs(x)))

    def softplus(x):
        return jnp.maximum(x, 0.0) + jnp.log(1.0 + jnp.exp(-jnp.abs(x)))

    p_int = jnp.sum(rr_ref[0], axis=-1)
    n_int = jnp.sum(rr_ref[1], axis=-1)
    p_pop = jnp.sum(rr_ref[2], axis=-1)
    n_pop = jnp.sum(rr_ref[3], axis=-1)
    mf = mm_ref[...]
    loss_int = -jnp.mean(mf * logsig(p_int - n_int))
    loss_pop = (-jnp.mean(mf * logsig(n_pop - p_pop))
                + jnp.mean((1.0 - mf) * logsig(p_pop - n_pop)))
    pop_p = softplus(qq_ref[0]) + softplus(qq_ref[2])
    pop_n = softplus(qq_ref[1]) + softplus(qq_ref[3])
    p_tot = p_int + p_pop
    n_tot = n_int + n_pop
    loss_tide = -jnp.mean(logsig(jnp.tanh(pop_p) * p_tot
                                 - jnp.tanh(pop_n) * n_tot))
    total = 0.5 * loss_int + 0.5 * loss_pop + 0.2 * loss_tide
    out_ref[...] = jnp.full((1, 1), total, jnp.float32)


@jax.jit
def kernel(embeddings_int, embeddings_pop, q, b, user, item_p, item_n, mask,
           edge_index):
    emb2 = jnp.stack([embeddings_int, embeddings_pop])
    pad = jnp.full((2, _EP - _E), _N, jnp.int32)
    ep = jnp.concatenate([edge_index.astype(jnp.int32), pad], axis=1)
    srcp = ep[0]
    dstp = ep[1]
    uf = user.reshape(-1).astype(jnp.int32)
    ipf = (item_p.reshape(-1) + _N_USER).astype(jnp.int32)
    inf_ = (item_n.reshape(-1) + _N_USER).astype(jnp.int32)

    r4, qb4, _, _ = _make_sc_call()(emb2, srcp, dstp, uf, ipf, inf_, q, b)

    rr = r4.reshape(4, 128, 128, 16)
    qq = qb4.reshape(4, 128, 128)
    mm = mask.reshape(-1).astype(jnp.float32).reshape(128, 128)
    out = pl.pallas_call(
        _loss_body,
        out_shape=jax.ShapeDtypeStruct((1, 1), jnp.float32),
    )(rr, qq, mm)
    return out[0, 0]


# speedup vs baseline: 5.4069x; 1.0439x over previous
import jax
import jax.numpy as jnp
from jax import lax
from jax.experimental import pallas as pl
from jax.experimental.pallas import tpu as pltpu
from jax.experimental.pallas import tpu_sc as plsc

_N_USER = 6000
_N_ITEM = 4000
_N = 10000
_NP = 10240
_D = 128
_E = 320000
_EC = 64
_BLK = 3
_EDGE_CHUNKS = 321
_EP = 328704
_ROWS_PER_TILE = 640
_TRI = 16384
_TRI_CHUNKS = 16


def _rsqrt16(x):
    i = lax.bitcast_convert_type(x, jnp.int32)
    i = jnp.int32(0x5F3759DF) - (i >> 1)
    y = lax.bitcast_convert_type(i, jnp.float32)
    for _ in range(3):
        y = y * (jnp.float32(1.5) - jnp.float32(0.5) * x * y * y)
    return y


def _splat(ref, pos):
    return plsc.load_gather(ref, [jnp.full((16,), pos, jnp.int32)])


def _sc_body(emb2, srcp, dstp, uf, ipf, inf_, q_hbm, b_hbm,
             r4, qb4, g2,
             acc_sh, degs_sh, degd_sh,
             big_v, a_v, b_v, esrc_v, edst_v, tidx_v, qidx_v,
             rout_v, qbout_v, zrow_v, ones_v,
             sem0, sem1, sem2):
    c = lax.axis_index("c")
    s = lax.axis_index("s")
    r0 = s * _ROWS_PER_TILE
    toff = s * (_EDGE_CHUNKS * _EC)
    zero16 = jnp.zeros((16,), jnp.float32)
    one16 = jnp.ones((16,), jnp.float32)

    for r in range(16):
        for j in range(8):
            zrow_v[r, pl.ds(16 * j, 16)] = zero16
    for j in range(4):
        ones_v[pl.ds(16 * j, 16)] = one16

    @pl.loop(0, _ROWS_PER_TILE // 16)
    def _zero_deg(i):
        a_v[pl.ds(i * 16, 16)] = zero16
        b_v[pl.ds(i * 16, 16)] = zero16

    pltpu.sync_copy(a_v, degs_sh.at[pl.ds(r0, _ROWS_PER_TILE)])
    pltpu.sync_copy(b_v, degd_sh.at[pl.ds(r0, _ROWS_PER_TILE)])

    @pl.loop(0, _ROWS_PER_TILE // 16)
    def _zacc(i):
        pltpu.sync_copy(zrow_v, acc_sh.at[pl.ds(r0 + i * 16, 16)])

    plsc.subcore_barrier()

    def _load_idx_block(bk):
        off = toff + bk * (_BLK * _EC)
        hi = []
        for j in range(_BLK):
            hi.append(pltpu.async_copy(
                srcp.at[pl.ds(off + j * _EC, _EC)], esrc_v.at[j], sem2))
            hi.append(pltpu.async_copy(
                dstp.at[pl.ds(off + j * _EC, _EC)], edst_v.at[j], sem2))
        for h in hi:
            h.wait()

    @pl.loop(0, _EDGE_CHUNKS // _BLK)
    def _hist(bk):
        _load_idx_block(bk)
        hs = []
        for j in range(_BLK):
            hs.append(pltpu.async_copy(
                ones_v, degs_sh.at[esrc_v.at[j]], sem1, add=True))
            hs.append(pltpu.async_copy(
                ones_v, degd_sh.at[edst_v.at[j]], sem1, add=True))
        for h in hs:
            h.wait()

    plsc.subcore_barrier()

    pltpu.sync_copy(degs_sh.at[pl.ds(r0, _ROWS_PER_TILE)], a_v)
    pltpu.sync_copy(degd_sh.at[pl.ds(r0, _ROWS_PER_TILE)], b_v)

    @pl.loop(0, _ROWS_PER_TILE // 16)
    def _rs(i):
        x = a_v[pl.ds(i * 16, 16)]
        a_v[pl.ds(i * 16, 16)] = _rsqrt16(jnp.maximum(x, jnp.float32(1.0)))
        y = b_v[pl.ds(i * 16, 16)]
        b_v[pl.ds(i * 16, 16)] = _rsqrt16(jnp.maximum(y, jnp.float32(1.0)))

    nreal = jnp.minimum(_ROWS_PER_TILE, jnp.maximum(0, _N - r0))
    nsl = nreal // 64
    nrem = (nreal - nsl * 64) // 16

    def _scale_rows(i, nrows, src_buf, dst_buf):
        for r in range(nrows):
            av = _splat(a_v, i * 64 + r)
            for j in range(8):
                big_v[dst_buf, r, pl.ds(16 * j, 16)] = (
                    big_v[src_buf, r, pl.ds(16 * j, 16)] * av)

    @pl.loop(0, nsl)
    def _g0(i):
        rr = r0 + i * 64
        pltpu.sync_copy(emb2.at[c, pl.ds(rr, 64)], big_v.at[0])
        _scale_rows(i, 64, 0, 1)
        pltpu.sync_copy(big_v.at[1], g2.at[c, pl.ds(rr, 64)])

    @pl.loop(0, nrem)
    def _g0r(i):
        rr = r0 + nsl * 64 + i * 16
        pltpu.sync_copy(emb2.at[c, pl.ds(rr, 16)], big_v.at[0, pl.ds(0, 16)])
        for r in range(16):
            av = _splat(a_v, nsl * 64 + i * 16 + r)
            for j in range(8):
                big_v[1, r, pl.ds(16 * j, 16)] = (
                    big_v[0, r, pl.ds(16 * j, 16)] * av)
        pltpu.sync_copy(big_v.at[1, pl.ds(0, 16)], g2.at[c, pl.ds(rr, 16)])

    @pl.when(s == 15)
    def _():
        @pl.loop(0, (_NP - _N) // 16)
        def _zpad(i):
            pltpu.sync_copy(zrow_v, g2.at[c, pl.ds(_N + i * 16, 16)])

    plsc.subcore_barrier()

    def _edge_pass():
        @pl.loop(0, _EDGE_CHUNKS // _BLK)
        def _edges(bk):
            _load_idx_block(bk)
            hg = [pltpu.async_copy(g2.at[c].at[esrc_v.at[j]],
                                   big_v.at[j], sem0)
                  for j in range(_BLK)]
            for h in hg:
                h.wait()
            hsct = [pltpu.async_copy(big_v.at[j],
                                     acc_sh.at[edst_v.at[j]], sem1, add=True)
                    for j in range(_BLK)]
            for h in hsct:
                h.wait()

    _edge_pass()
    plsc.subcore_barrier()

    def _np1_rows(i, nrows):
        for r in range(nrows):
            abv = _splat(a_v, i * 64 + r) * _splat(b_v, i * 64 + r)
            for j in range(8):
                big_v[1, r, pl.ds(16 * j, 16)] = (
                    big_v[0, r, pl.ds(16 * j, 16)] * abv)

    @pl.loop(0, nsl)
    def _np1(i):
        rr = r0 + i * 64
        pltpu.sync_copy(acc_sh.at[pl.ds(rr, 64)], big_v.at[0])
        _np1_rows(i, 64)
        pltpu.sync_copy(big_v.at[1], g2.at[c, pl.ds(rr, 64)])

    @pl.loop(0, nrem)
    def _np1r(i):
        rr = r0 + nsl * 64 + i * 16
        pltpu.sync_copy(acc_sh.at[pl.ds(rr, 16)], big_v.at[0, pl.ds(0, 16)])
        for r in range(16):
            k = nsl * 64 + i * 16 + r
            abv = _splat(a_v, k) * _splat(b_v, k)
            for j in range(8):
                big_v[1, r, pl.ds(16 * j, 16)] = (
                    big_v[0, r, pl.ds(16 * j, 16)] * abv)
        pltpu.sync_copy(big_v.at[1, pl.ds(0, 16)], g2.at[c, pl.ds(rr, 16)])

    plsc.subcore_barrier()

    _edge_pass()
    plsc.subcore_barrier()

    third = jnp.float32(1.0 / 3.0)

    def _np2_rows(i, nrows):
        for r in range(nrows):
            bv = _splat(b_v, i * 64 + r)
            for j in range(8):
                big_v[1, r, pl.ds(16 * j, 16)] = (
                    big_v[1, r, pl.ds(16 * j, 16)]
                    + big_v[0, r, pl.ds(16 * j, 16)] * bv) * third

    @pl.loop(0, nsl)
    def _np2(i):
        rr = r0 + i * 64
        h0 = pltpu.async_copy(acc_sh.at[pl.ds(rr, 64)], big_v.at[0], sem0)
        h1 = pltpu.async_copy(emb2.at[c, pl.ds(rr, 64)], big_v.at[1], sem2)
        h0.wait()
        h1.wait()
        _np2_rows(i, 64)
        pltpu.sync_copy(big_v.at[1], g2.at[c, pl.ds(rr, 64)])

    @pl.loop(0, nrem)
    def _np2r(i):
        rr = r0 + nsl * 64 + i * 16
        pltpu.sync_copy(acc_sh.at[pl.ds(rr, 16)], big_v.at[0, pl.ds(0, 16)])
        pltpu.sync_copy(emb2.at[c, pl.ds(rr, 16)], big_v.at[1, pl.ds(0, 16)])
        for r in range(16):
            bv = _splat(b_v, nsl * 64 + i * 16 + r)
            for j in range(8):
                big_v[1, r, pl.ds(16 * j, 16)] = (
                    big_v[1, r, pl.ds(16 * j, 16)]
                    + big_v[0, r, pl.ds(16 * j, 16)] * bv) * third
        pltpu.sync_copy(big_v.at[1, pl.ds(0, 16)], g2.at[c, pl.ds(rr, 16)])

    plsc.subcore_barrier()

    @pl.loop(0, _TRI_CHUNKS)
    def _dots(ch):
        t0 = s * (_TRI_CHUNKS * 64) + ch * 64
        pltpu.sync_copy(uf.at[pl.ds(t0, 64)], tidx_v.at[0])
        pltpu.sync_copy(ipf.at[pl.ds(t0, 64)], tidx_v.at[1])
        pltpu.sync_copy(inf_.at[pl.ds(t0, 64)], tidx_v.at[2])
        hg = [pltpu.async_copy(g2.at[c].at[tidx_v.at[z]], big_v.at[z], sem0)
              for z in range(3)]

        @pl.when(c == 1)
        def _():
            for j in range(4):
                sl = pl.ds(16 * j, 16)
                qidx_v[0, sl] = tidx_v[1, sl] - _N_USER
                qidx_v[1, sl] = tidx_v[2, sl] - _N_USER
            hq = [pltpu.async_copy(q_hbm.at[qidx_v.at[0]], qbout_v.at[0],
                                   sem2),
                  pltpu.async_copy(q_hbm.at[qidx_v.at[1]], qbout_v.at[1],
                                   sem2),
                  pltpu.async_copy(b_hbm.at[qidx_v.at[0]], qbout_v.at[2],
                                   sem2),
                  pltpu.async_copy(b_hbm.at[qidx_v.at[1]], qbout_v.at[3],
                                   sem2)]
            for h in hq:
                h.wait()
            for z in range(4):
                pltpu.sync_copy(qbout_v.at[z], qb4.at[z, pl.ds(t0, 64)])

        for h in hg:
            h.wait()

        @pl.loop(0, 64)
        def _row(r):
            pacc = jnp.zeros((16,), jnp.float32)
            nacc = jnp.zeros((16,), jnp.float32)
            for j in range(8):
                u = big_v[0, r, pl.ds(16 * j, 16)]
                pacc = pacc + u * big_v[1, r, pl.ds(16 * j, 16)]
                nacc = nacc + u * big_v[2, r, pl.ds(16 * j, 16)]
            rout_v[0, r, pl.ds(0, 16)] = pacc
            rout_v[1, r, pl.ds(0, 16)] = nacc

        pltpu.sync_copy(rout_v.at[0], r4.at[2 * c, pl.ds(t0, 64)])
        pltpu.sync_copy(rout_v.at[1], r4.at[2 * c + 1, pl.ds(t0, 64)])


def _make_sc_call():
    mesh = plsc.VectorSubcoreMesh(core_axis_name="c", subcore_axis_name="s")
    out_type = (
        jax.ShapeDtypeStruct((4, _TRI, 16), jnp.float32),
        jax.ShapeDtypeStruct((4, _TRI), jnp.float32),
        jax.ShapeDtypeStruct((2, _NP, _D), jnp.float32),
    )
    scratch = [
        pltpu.VMEM_SHARED((_NP, _D), jnp.float32),
        pltpu.VMEM_SHARED((_NP,), jnp.float32),
        pltpu.VMEM_SHARED((_NP,), jnp.float32),
        pltpu.VMEM((_BLK, _EC, _D), jnp.float32),
        pltpu.VMEM((_ROWS_PER_TILE,), jnp.float32),
        pltpu.VMEM((_ROWS_PER_TILE,), jnp.float32),
        pltpu.VMEM((_BLK, _EC), jnp.int32),
        pltpu.VMEM((_BLK, _EC), jnp.int32),
        pltpu.VMEM((3, _EC), jnp.int32),
        pltpu.VMEM((2, _EC), jnp.int32),
        pltpu.VMEM((2, _EC, 16), jnp.float32),
        pltpu.VMEM((4, _EC), jnp.float32),
        pltpu.VMEM((16, _D), jnp.float32),
        pltpu.VMEM((_EC,), jnp.float32),
        pltpu.SemaphoreType.DMA,
        pltpu.SemaphoreType.DMA,
        pltpu.SemaphoreType.DMA,
    ]
    return pl.kernel(_sc_body, out_type=out_type, mesh=mesh,
                     scratch_types=scratch,
                     compiler_params=pltpu.CompilerParams(
                         needs_layout_passes=False))


def _loss_body(rr_ref, qq_ref, mm_ref, out_ref):
    def logsig(x):
        return jnp.minimum(x, 0.0) - jnp.log(1.0 + jnp.exp(-jnp.abs(x)))

    def softplus(x):
        return jnp.maximum(x, 0.0) + jnp.log(1.0 + jnp.exp(-jnp.abs(x)))

    p_int = jnp.sum(rr_ref[0], axis=-1)
    n_int = jnp.sum(rr_ref[1], axis=-1)
    p_pop = jnp.sum(rr_ref[2], axis=-1)
    n_pop = jnp.sum(rr_ref[3], axis=-1)
    mf = mm_ref[...]
    loss_int = -jnp.mean(mf * logsig(p_int - n_int))
    loss_pop = (-jnp.mean(mf * logsig(n_pop - p_pop))
                + jnp.mean((1.0 - mf) * logsig(p_pop - n_pop)))
    pop_p = softplus(qq_ref[0]) + softplus(qq_ref[2])
    pop_n = softplus(qq_ref[1]) + softplus(qq_ref[3])
    p_tot = p_int + p_pop
    n_tot = n_int + n_pop
    loss_tide = -jnp.mean(logsig(jnp.tanh(pop_p) * p_tot
                                 - jnp.tanh(pop_n) * n_tot))
    total = 0.5 * loss_int + 0.5 * loss_pop + 0.2 * loss_tide
    out_ref[...] = jnp.full((1, 1), total, jnp.float32)


@jax.jit
def kernel(embeddings_int, embeddings_pop, q, b, user, item_p, item_n, mask,
           edge_index):
    emb2 = jnp.stack([embeddings_int, embeddings_pop])
    pad = jnp.full((2, _EP - _E), _N, jnp.int32)
    ep = jnp.concatenate([edge_index.astype(jnp.int32), pad], axis=1)
    srcp = ep[0]
    dstp = ep[1]
    uf = user.reshape(-1).astype(jnp.int32)
    ipf = (item_p.reshape(-1) + _N_USER).astype(jnp.int32)
    inf_ = (item_n.reshape(-1) + _N_USER).astype(jnp.int32)

    r4, qb4, _ = _make_sc_call()(emb2, srcp, dstp, uf, ipf, inf_, q, b)

    rr = r4.reshape(4, 128, 128, 16)
    qq = qb4.reshape(4, 128, 128)
    mm = mask.reshape(-1).astype(jnp.float32).reshape(128, 128)
    out = pl.pallas_call(
        _loss_body,
        out_shape=jax.ShapeDtypeStruct((1, 1), jnp.float32),
    )(rr, qq, mm)
    return out[0, 0]


# speedup vs baseline: 5.7481x; 1.0631x over previous
import jax
import jax.numpy as jnp
from jax import lax
from jax.experimental import pallas as pl
from jax.experimental.pallas import tpu as pltpu
from jax.experimental.pallas import tpu_sc as plsc

_N_USER = 6000
_N_ITEM = 4000
_N = 10000
_NP = 10240
_D = 128
_E = 320000
_EC = 64
_BLK = 3
_EDGE_CHUNKS = 321
_EP = 328704
_ROWS_PER_TILE = 640
_TRI = 16384
_TRI_CHUNKS = 16


def _rsqrt16(x):
    i = lax.bitcast_convert_type(x, jnp.int32)
    i = jnp.int32(0x5F3759DF) - (i >> 1)
    y = lax.bitcast_convert_type(i, jnp.float32)
    for _ in range(3):
        y = y * (jnp.float32(1.5) - jnp.float32(0.5) * x * y * y)
    return y


def _splat(ref, pos):
    return plsc.load_gather(ref, [jnp.full((16,), pos, jnp.int32)])


def _sc_body(emb2, srcp, dstp, uf, ipf, inf_, q_hbm, b_hbm,
             r4, qb4, g2,
             acc_sh, degs_sh, degd_sh,
             big_v, a_v, b_v, esrc_v, edst_v, tidx_v, qidx_v,
             rout_v, qbout_v, zrow_v, ones_v,
             sem0, sem1, sem2):
    c = lax.axis_index("c")
    s = lax.axis_index("s")
    r0 = s * _ROWS_PER_TILE
    toff = s * (_EDGE_CHUNKS * _EC)
    zero16 = jnp.zeros((16,), jnp.float32)
    one16 = jnp.ones((16,), jnp.float32)

    for r in range(16):
        for j in range(8):
            zrow_v[r, pl.ds(16 * j, 16)] = zero16
    for j in range(4):
        ones_v[pl.ds(16 * j, 16)] = one16

    @pl.loop(0, _ROWS_PER_TILE // 16)
    def _zero_deg(i):
        a_v[pl.ds(i * 16, 16)] = zero16
        b_v[pl.ds(i * 16, 16)] = zero16

    pltpu.sync_copy(a_v, degs_sh.at[pl.ds(r0, _ROWS_PER_TILE)])
    pltpu.sync_copy(b_v, degd_sh.at[pl.ds(r0, _ROWS_PER_TILE)])

    @pl.loop(0, _ROWS_PER_TILE // 16)
    def _zacc(i):
        pltpu.sync_copy(zrow_v, acc_sh.at[pl.ds(r0 + i * 16, 16)])

    plsc.subcore_barrier()

    def _load_idx_block(bk, p):
        off = toff + bk * (_BLK * _EC)
        hi = []
        for j in range(_BLK):
            hi.append(pltpu.async_copy(
                srcp.at[pl.ds(off + j * _EC, _EC)], esrc_v.at[p, j], sem2))
            hi.append(pltpu.async_copy(
                dstp.at[pl.ds(off + j * _EC, _EC)], edst_v.at[p, j], sem2))
        for h in hi:
            h.wait()

    @pl.loop(0, _EDGE_CHUNKS // _BLK)
    def _hist(bk):
        _load_idx_block(bk, 0)
        hs = []
        for j in range(_BLK):
            hs.append(pltpu.async_copy(
                ones_v, degs_sh.at[esrc_v.at[0, j]], sem1, add=True))
            hs.append(pltpu.async_copy(
                ones_v, degd_sh.at[edst_v.at[0, j]], sem1, add=True))
        for h in hs:
            h.wait()

    plsc.subcore_barrier()

    pltpu.sync_copy(degs_sh.at[pl.ds(r0, _ROWS_PER_TILE)], a_v)
    pltpu.sync_copy(degd_sh.at[pl.ds(r0, _ROWS_PER_TILE)], b_v)

    @pl.loop(0, _ROWS_PER_TILE // 16)
    def _rs(i):
        x = a_v[pl.ds(i * 16, 16)]
        a_v[pl.ds(i * 16, 16)] = _rsqrt16(jnp.maximum(x, jnp.float32(1.0)))
        y = b_v[pl.ds(i * 16, 16)]
        b_v[pl.ds(i * 16, 16)] = _rsqrt16(jnp.maximum(y, jnp.float32(1.0)))

    nreal = jnp.minimum(_ROWS_PER_TILE, jnp.maximum(0, _N - r0))
    nsl = nreal // 64
    nrem = (nreal - nsl * 64) // 16

    def _scale_rows(i, nrows, src_buf, dst_buf):
        for r in range(nrows):
            av = _splat(a_v, i * 64 + r)
            for j in range(8):
                big_v[dst_buf, r, pl.ds(16 * j, 16)] = (
                    big_v[src_buf, r, pl.ds(16 * j, 16)] * av)

    @pl.loop(0, nsl)
    def _g0(i):
        rr = r0 + i * 64
        pltpu.sync_copy(emb2.at[c, pl.ds(rr, 64)], big_v.at[0])
        _scale_rows(i, 64, 0, 1)
        pltpu.sync_copy(big_v.at[1], g2.at[c, pl.ds(rr, 64)])

    @pl.loop(0, nrem)
    def _g0r(i):
        rr = r0 + nsl * 64 + i * 16
        pltpu.sync_copy(emb2.at[c, pl.ds(rr, 16)], big_v.at[0, pl.ds(0, 16)])
        for r in range(16):
            av = _splat(a_v, nsl * 64 + i * 16 + r)
            for j in range(8):
                big_v[1, r, pl.ds(16 * j, 16)] = (
                    big_v[0, r, pl.ds(16 * j, 16)] * av)
        pltpu.sync_copy(big_v.at[1, pl.ds(0, 16)], g2.at[c, pl.ds(rr, 16)])

    @pl.when(s == 15)
    def _():
        @pl.loop(0, (_NP - _N) // 16)
        def _zpad(i):
            pltpu.sync_copy(zrow_v, g2.at[c, pl.ds(_N + i * 16, 16)])

    plsc.subcore_barrier()

    nblk = _EDGE_CHUNKS // _BLK

    def _edge_pass():
        _load_idx_block(0, 0)

        @pl.loop(0, nblk)
        def _edges(bk):
            p = bk % 2
            hg = [pltpu.async_copy(g2.at[c].at[esrc_v.at[p, j]],
                                   big_v.at[j], sem0)
                  for j in range(_BLK)]
            for h in hg:
                h.wait()
            hsct = [pltpu.async_copy(big_v.at[j],
                                     acc_sh.at[edst_v.at[p, j]],
                                     sem1, add=True)
                    for j in range(_BLK)]

            @pl.when(bk + 1 < nblk)
            def _():
                _load_idx_block(bk + 1, (bk + 1) % 2)

            for h in hsct:
                h.wait()

    _edge_pass()
    plsc.subcore_barrier()

    def _np1_rows(i, nrows):
        for r in range(nrows):
            abv = _splat(a_v, i * 64 + r) * _splat(b_v, i * 64 + r)
            for j in range(8):
                big_v[1, r, pl.ds(16 * j, 16)] = (
                    big_v[0, r, pl.ds(16 * j, 16)] * abv)

    @pl.loop(0, nsl)
    def _np1(i):
        rr = r0 + i * 64
        pltpu.sync_copy(acc_sh.at[pl.ds(rr, 64)], big_v.at[0])
        _np1_rows(i, 64)
        pltpu.sync_copy(big_v.at[1], g2.at[c, pl.ds(rr, 64)])

    @pl.loop(0, nrem)
    def _np1r(i):
        rr = r0 + nsl * 64 + i * 16
        pltpu.sync_copy(acc_sh.at[pl.ds(rr, 16)], big_v.at[0, pl.ds(0, 16)])
        for r in range(16):
            k = nsl * 64 + i * 16 + r
            abv = _splat(a_v, k) * _splat(b_v, k)
            for j in range(8):
                big_v[1, r, pl.ds(16 * j, 16)] = (
                    big_v[0, r, pl.ds(16 * j, 16)] * abv)
        pltpu.sync_copy(big_v.at[1, pl.ds(0, 16)], g2.at[c, pl.ds(rr, 16)])

    plsc.subcore_barrier()

    _edge_pass()
    plsc.subcore_barrier()

    third = jnp.float32(1.0 / 3.0)

    def _np2_rows(i, nrows):
        for r in range(nrows):
            bv = _splat(b_v, i * 64 + r)
            for j in range(8):
                big_v[1, r, pl.ds(16 * j, 16)] = (
                    big_v[1, r, pl.ds(16 * j, 16)]
                    + big_v[0, r, pl.ds(16 * j, 16)] * bv) * third

    @pl.loop(0, nsl)
    def _np2(i):
        rr = r0 + i * 64
        h0 = pltpu.async_copy(acc_sh.at[pl.ds(rr, 64)], big_v.at[0], sem0)
        h1 = pltpu.async_copy(emb2.at[c, pl.ds(rr, 64)], big_v.at[1], sem2)
        h0.wait()
        h1.wait()
        _np2_rows(i, 64)
        pltpu.sync_copy(big_v.at[1], g2.at[c, pl.ds(rr, 64)])

    @pl.loop(0, nrem)
    def _np2r(i):
        rr = r0 + nsl * 64 + i * 16
        pltpu.sync_copy(acc_sh.at[pl.ds(rr, 16)], big_v.at[0, pl.ds(0, 16)])
        pltpu.sync_copy(emb2.at[c, pl.ds(rr, 16)], big_v.at[1, pl.ds(0, 16)])
        for r in range(16):
            bv = _splat(b_v, nsl * 64 + i * 16 + r)
            for j in range(8):
                big_v[1, r, pl.ds(16 * j, 16)] = (
                    big_v[1, r, pl.ds(16 * j, 16)]
                    + big_v[0, r, pl.ds(16 * j, 16)] * bv) * third
        pltpu.sync_copy(big_v.at[1, pl.ds(0, 16)], g2.at[c, pl.ds(rr, 16)])

    plsc.subcore_barrier()

    @pl.loop(0, _TRI_CHUNKS)
    def _dots(ch):
        t0 = s * (_TRI_CHUNKS * 64) + ch * 64
        ht = [pltpu.async_copy(uf.at[pl.ds(t0, 64)], tidx_v.at[0], sem2),
              pltpu.async_copy(ipf.at[pl.ds(t0, 64)], tidx_v.at[1], sem2),
              pltpu.async_copy(inf_.at[pl.ds(t0, 64)], tidx_v.at[2], sem2)]
        for h in ht:
            h.wait()
        hg = [pltpu.async_copy(g2.at[c].at[tidx_v.at[z]], big_v.at[z], sem0)
              for z in range(3)]

        @pl.when(c == 1)
        def _():
            for j in range(4):
                sl = pl.ds(16 * j, 16)
                qidx_v[0, sl] = tidx_v[1, sl] - _N_USER
                qidx_v[1, sl] = tidx_v[2, sl] - _N_USER
            hq = [pltpu.async_copy(q_hbm.at[qidx_v.at[0]], qbout_v.at[0],
                                   sem2),
                  pltpu.async_copy(q_hbm.at[qidx_v.at[1]], qbout_v.at[1],
                                   sem2),
                  pltpu.async_copy(b_hbm.at[qidx_v.at[0]], qbout_v.at[2],
                                   sem2),
                  pltpu.async_copy(b_hbm.at[qidx_v.at[1]], qbout_v.at[3],
                                   sem2)]
            for h in hq:
                h.wait()
            for z in range(4):
                pltpu.sync_copy(qbout_v.at[z], qb4.at[z, pl.ds(t0, 64)])

        for h in hg:
            h.wait()

        @pl.loop(0, 64)
        def _row(r):
            pacc = jnp.zeros((16,), jnp.float32)
            nacc = jnp.zeros((16,), jnp.float32)
            for j in range(8):
                u = big_v[0, r, pl.ds(16 * j, 16)]
                pacc = pacc + u * big_v[1, r, pl.ds(16 * j, 16)]
                nacc = nacc + u * big_v[2, r, pl.ds(16 * j, 16)]
            rout_v[0, r, pl.ds(0, 16)] = pacc
            rout_v[1, r, pl.ds(0, 16)] = nacc

        pltpu.sync_copy(rout_v.at[0], r4.at[2 * c, pl.ds(t0, 64)])
        pltpu.sync_copy(rout_v.at[1], r4.at[2 * c + 1, pl.ds(t0, 64)])


def _make_sc_call():
    mesh = plsc.VectorSubcoreMesh(core_axis_name="c", subcore_axis_name="s")
    out_type = (
        jax.ShapeDtypeStruct((4, _TRI, 16), jnp.float32),
        jax.ShapeDtypeStruct((4, _TRI), jnp.float32),
        jax.ShapeDtypeStruct((2, _NP, _D), jnp.float32),
    )
    scratch = [
        pltpu.VMEM_SHARED((_NP, _D), jnp.float32),
        pltpu.VMEM_SHARED((_NP,), jnp.float32),
        pltpu.VMEM_SHARED((_NP,), jnp.float32),
        pltpu.VMEM((_BLK, _EC, _D), jnp.float32),
        pltpu.VMEM((_ROWS_PER_TILE,), jnp.float32),
        pltpu.VMEM((_ROWS_PER_TILE,), jnp.float32),
        pltpu.VMEM((2, _BLK, _EC), jnp.int32),
        pltpu.VMEM((2, _BLK, _EC), jnp.int32),
        pltpu.VMEM((3, _EC), jnp.int32),
        pltpu.VMEM((2, _EC), jnp.int32),
        pltpu.VMEM((2, _EC, 16), jnp.float32),
        pltpu.VMEM((4, _EC), jnp.float32),
        pltpu.VMEM((16, _D), jnp.float32),
        pltpu.VMEM((_EC,), jnp.float32),
        pltpu.SemaphoreType.DMA,
        pltpu.SemaphoreType.DMA,
        pltpu.SemaphoreType.DMA,
    ]
    return pl.kernel(_sc_body, out_type=out_type, mesh=mesh,
                     scratch_types=scratch,
                     compiler_params=pltpu.CompilerParams(
                         needs_layout_passes=False))


def _loss_body(rr_ref, qq_ref, mm_ref, out_ref):
    def logsig(x):
        return jnp.minimum(x, 0.0) - jnp.log(1.0 + jnp.exp(-jnp.abs(x)))

    def softplus(x):
        return jnp.maximum(x, 0.0) + jnp.log(1.0 + jnp.exp(-jnp.abs(x)))

    p_int = jnp.sum(rr_ref[0], axis=-1)
    n_int = jnp.sum(rr_ref[1], axis=-1)
    p_pop = jnp.sum(rr_ref[2], axis=-1)
    n_pop = jnp.sum(rr_ref[3], axis=-1)
    mf = mm_ref[...]
    loss_int = -jnp.mean(mf * logsig(p_int - n_int))
    loss_pop = (-jnp.mean(mf * logsig(n_pop - p_pop))
                + jnp.mean((1.0 - mf) * logsig(p_pop - n_pop)))
    pop_p = softplus(qq_ref[0]) + softplus(qq_ref[2])
    pop_n = softplus(qq_ref[1]) + softplus(qq_ref[3])
    p_tot = p_int + p_pop
    n_tot = n_int + n_pop
    loss_tide = -jnp.mean(logsig(jnp.tanh(pop_p) * p_tot
                                 - jnp.tanh(pop_n) * n_tot))
    total = 0.5 * loss_int + 0.5 * loss_pop + 0.2 * loss_tide
    out_ref[...] = jnp.full((1, 1), total, jnp.float32)


@jax.jit
def kernel(embeddings_int, embeddings_pop, q, b, user, item_p, item_n, mask,
           edge_index):
    emb2 = jnp.stack([embeddings_int, embeddings_pop])
    pad = jnp.full((2, _EP - _E), _N, jnp.int32)
    ep = jnp.concatenate([edge_index.astype(jnp.int32), pad], axis=1)
    srcp = ep[0]
    dstp = ep[1]
    uf = user.reshape(-1).astype(jnp.int32)
    ipf = (item_p.reshape(-1) + _N_USER).astype(jnp.int32)
    inf_ = (item_n.reshape(-1) + _N_USER).astype(jnp.int32)

    r4, qb4, _ = _make_sc_call()(emb2, srcp, dstp, uf, ipf, inf_, q, b)

    rr = r4.reshape(4, 128, 128, 16)
    qq = qb4.reshape(4, 128, 128)
    mm = mask.reshape(-1).astype(jnp.float32).reshape(128, 128)
    out = pl.pallas_call(
        _loss_body,
        out_shape=jax.ShapeDtypeStruct((1, 1), jnp.float32),
    )(rr, qq, mm)
    return out[0, 0]
